# baseline (speedup 1.0000x reference)
import functools

import jax
import jax.numpy as jnp
from jax import lax
from jax.experimental import pallas as pl
from jax.experimental.pallas import tpu as pltpu
from jax.experimental.pallas import tpu_sc as plsc

_NC = 2
_NS = 16
_NW = _NC * _NS
_CH = 128

_N = 10000
_E = 320000
_D = 128
_NPAD = 10112
_ROWS_PER_TILE = _NPAD // _NS
_SUP = 16
_NCHUNK = 80
_NSUP = _NCHUNK // _SUP
_EPT = _NCHUNK * _CH
_EPAD = _EPT * _NW

_mesh = plsc.VectorSubcoreMesh(core_axis_name="c", subcore_axis_name="s")


def _zero_rows(rows_v, nrow, ncol16):
    def body(i, carry):
        for k in range(ncol16):
            rows_v[i, pl.ds(k * 16, 16)] = jnp.zeros((16,), jnp.float32)
        return carry
    lax.fori_loop(0, nrow, body, 0)


def _spmem_zero_and_barrier(zsrc_v, shared, s, width):
    base = s * _ROWS_PER_TILE
    off = 0
    while off < _ROWS_PER_TILE:
        m = min(_CH, _ROWS_PER_TILE - off)
        pltpu.sync_copy(zsrc_v.at[pl.ds(0, m)], shared.at[pl.ds(base + off, m)])
        off += m
    plsc.subcore_barrier()


def _spmem_writeout(shared, rows_v, out_slice, s):
    base = s * _ROWS_PER_TILE
    off = 0
    while off < _ROWS_PER_TILE:
        m = min(_CH, _ROWS_PER_TILE - off)
        pltpu.sync_copy(shared.at[pl.ds(base + off, m)], rows_v.at[pl.ds(0, m)])
        pltpu.sync_copy(rows_v.at[pl.ds(0, m)], out_slice.at[pl.ds(base + off, m)])
        off += m


@functools.partial(
    pl.kernel,
    out_type=jax.ShapeDtypeStruct((_NC, _NPAD, _D), jnp.float32),
    mesh=_mesh,
    scratch_types=[
        pltpu.VMEM((_NCHUNK, _CH), jnp.int32),
        pltpu.VMEM((_NCHUNK, _CH), jnp.int32),
        pltpu.VMEM((_CH, _D), jnp.float32),
        pltpu.VMEM_SHARED((_NPAD, _D), jnp.float32),
    ],
)
def _sc_agg(h_hbm, src_hbm, dst_hbm, out_hbm, src_v, dst_v, rows, acc_sh):
    c = lax.axis_index("c")
    s = lax.axis_index("s")
    wid = c * _NS + s
    pltpu.sync_copy(src_hbm.at[wid], src_v)
    pltpu.sync_copy(dst_hbm.at[wid], dst_v)
    _zero_rows(rows, _CH, _D // 16)
    _spmem_zero_and_barrier(rows, acc_sh, s, _D)

    def body(j, carry):
        pltpu.sync_copy(h_hbm.at[src_v.at[j]], rows)
        pltpu.sync_copy(rows, acc_sh.at[dst_v.at[j]], add=True)
        return carry
    lax.fori_loop(0, _NCHUNK, body, 0)

    plsc.subcore_barrier()
    _spmem_writeout(acc_sh, rows, out_hbm.at[c], s)


@functools.partial(
    pl.kernel,
    out_type=jax.ShapeDtypeStruct((_NC, _NPAD, _D), jnp.float32),
    mesh=_mesh,
    scratch_types=[
        pltpu.VMEM((_NCHUNK, _CH), jnp.int32),
        pltpu.VMEM((_CH, _D), jnp.float32),
        pltpu.VMEM((_CH, _D), jnp.float32),
        pltpu.VMEM_SHARED((_NPAD, _D), jnp.float32),
    ],
)
def _sc_deg(dst_hbm, out_hbm, dst_v, ones_v, zero_v, acc_sh):
    c = lax.axis_index("c")
    s = lax.axis_index("s")
    wid = c * _NS + s
    pltpu.sync_copy(dst_hbm.at[wid], dst_v)

    def fill(i, carry):
        for k in range(_D // 16):
            ones_v[i, pl.ds(k * 16, 16)] = jnp.ones((16,), jnp.float32)
            zero_v[i, pl.ds(k * 16, 16)] = jnp.zeros((16,), jnp.float32)
        return carry
    lax.fori_loop(0, _CH, fill, 0)
    _spmem_zero_and_barrier(zero_v, acc_sh, s, _D)

    def body(j, carry):
        pltpu.sync_copy(ones_v, acc_sh.at[dst_v.at[j]], add=True)
        return carry
    lax.fori_loop(0, _NCHUNK, body, 0)

    plsc.subcore_barrier()
    _spmem_writeout(acc_sh, zero_v, out_hbm.at[c], s)


def _matmul(a, w):
    return lax.dot_general(
        a, w, (((1,), (0,)), ((), ())),
        precision=lax.Precision.HIGHEST,
        preferred_element_type=jnp.float32,
    )


_BM = 2000
_NBLK = _N // _BM
_DBM = 1264


def _invd_body(dd_ref, o_ref):
    deg = dd_ref[0, :, 0:1] + dd_ref[1, :, 0:1]
    o_ref[...] = jnp.broadcast_to(1.0 / jnp.maximum(deg, 1.0), o_ref.shape)


def _invdeg(dd):
    return pl.pallas_call(
        _invd_body,
        grid=(_NPAD // _DBM,),
        in_specs=[pl.BlockSpec((_NC, _DBM, _D), lambda j: (0, j, 0))],
        out_specs=pl.BlockSpec((_DBM, _D), lambda j: (j, 0)),
        out_shape=jax.ShapeDtypeStruct((_NPAD, _D), jnp.float32),
    )(dd)


def _linear_body(h_ref, pp_ref, iv_ref, wl_ref, bl_ref, wr_ref,
                 t_ref, st_ref, acc_ref):
    j = pl.program_id(0)
    agg = pp_ref[0] + pp_ref[1]
    mean = agg * iv_ref[...]
    t = _matmul(mean, wl_ref[...]) + _matmul(h_ref[...], wr_ref[...]) \
        + bl_ref[...]
    t_ref[...] = t

    @pl.when(j == 0)
    def _():
        acc_ref[...] = jnp.zeros_like(acc_ref)

    acc_ref[0:1, :] += jnp.sum(t, axis=0, keepdims=True)
    acc_ref[1:2, :] += jnp.sum(t * t, axis=0, keepdims=True)

    @pl.when(j == _NBLK - 1)
    def _():
        mu = acc_ref[0:1, :] * (1.0 / _N)
        st_ref[0:1, :] = mu
        st_ref[1:2, :] = acc_ref[1:2, :] * (1.0 / _N) - mu * mu


def _bn_body(t_ref, st_ref, g_ref, b_ref, wc_ref, bc_ref, o_ref, *, last):
    mu = st_ref[0:1, :]
    var = st_ref[1:2, :]
    y = (t_ref[...] - mu) * lax.rsqrt(var + 1e-5) * g_ref[...] + b_ref[...]
    y = jnp.maximum(y, 0.0)
    if last:
        o_ref[...] = _matmul(y, wc_ref[...]) + bc_ref[...]
    else:
        o_ref[...] = y


def _dense_layer(h, pp, iv, Wl, bl, Wr, g, b, Wc, bc, last):
    t, st = pl.pallas_call(
        _linear_body,
        grid=(_NBLK,),
        in_specs=[
            pl.BlockSpec((_BM, _D), lambda j: (j, 0)),
            pl.BlockSpec((_NC, _BM, _D), lambda j: (0, j, 0)),
            pl.BlockSpec((_BM, _D), lambda j: (j, 0)),
            pl.BlockSpec((_D, _D), lambda j: (0, 0)),
            pl.BlockSpec((1, _D), lambda j: (0, 0)),
            pl.BlockSpec((_D, _D), lambda j: (0, 0)),
        ],
        out_specs=[
            pl.BlockSpec((_BM, _D), lambda j: (j, 0)),
            pl.BlockSpec((2, _D), lambda j: (0, 0)),
        ],
        out_shape=[
            jax.ShapeDtypeStruct((_N, _D), jnp.float32),
            jax.ShapeDtypeStruct((2, _D), jnp.float32),
        ],
        scratch_shapes=[pltpu.VMEM((2, _D), jnp.float32)],
    )(h, pp, iv, Wl, bl.reshape(1, -1), Wr)

    return pl.pallas_call(
        functools.partial(_bn_body, last=last),
        grid=(_NBLK,),
        in_specs=[
            pl.BlockSpec((_BM, _D), lambda j: (j, 0)),
            pl.BlockSpec((2, _D), lambda j: (0, 0)),
            pl.BlockSpec((1, _D), lambda j: (0, 0)),
            pl.BlockSpec((1, _D), lambda j: (0, 0)),
            pl.BlockSpec((_D, _D), lambda j: (0, 0)),
            pl.BlockSpec((1, _D), lambda j: (0, 0)),
        ],
        out_specs=pl.BlockSpec((_BM, _D), lambda j: (j, 0)),
        out_shape=jax.ShapeDtypeStruct((_N, _D), jnp.float32),
    )(t, st, g.reshape(1, -1), b.reshape(1, -1), Wc, bc.reshape(1, -1))


def kernel(x, edge_index, Wl0, bl0, Wr0, g0, b0, Wl1, bl1, Wr1, g1, b1,
           Wl2, bl2, Wr2, g2, b2, Wc, bc):
    src = edge_index[0]
    dst = edge_index[1]
    pad = _EPAD - _E
    src3 = jnp.concatenate([src, jnp.zeros((pad,), jnp.int32)]).reshape(
        _NW, _NCHUNK, _CH)
    dst3 = jnp.concatenate([dst, jnp.full((pad,), _N, jnp.int32)]).reshape(
        _NW, _NCHUNK, _CH)

    dd = _sc_deg(dst3)
    iv = _invdeg(dd)

    c = Wc.shape[1]
    Wcp = jnp.zeros((_D, 128), jnp.float32).at[:, :c].set(Wc)
    bcp = jnp.zeros((128,), jnp.float32).at[:c].set(bc)

    h = x
    params = [(Wl0, bl0, Wr0, g0, b0), (Wl1, bl1, Wr1, g1, b1),
              (Wl2, bl2, Wr2, g2, b2)]
    for i, (Wl, bl, Wr, g, b) in enumerate(params):
        pp = _sc_agg(h, src3, dst3)
        h = _dense_layer(h, pp, iv, Wl, bl, Wr, g, b, Wcp, bcp,
                         last=(i == 2))
    return h[:, :c]


# speedup vs baseline: 1.4879x; 1.4879x over previous
import functools

import jax
import jax.numpy as jnp
from jax import lax
from jax.experimental import pallas as pl
from jax.experimental.pallas import tpu as pltpu
from jax.experimental.pallas import tpu_sc as plsc

_NC = 2
_NS = 16
_NW = _NC * _NS
_CH = 128

_N = 10000
_E = 320000
_D = 128
_NPAD = 10112
_ROWS_PER_TILE = _NPAD // _NS
_NCHUNK = 79
_EPT = _NCHUNK * _CH
_EPAD = _EPT * _NW

_mesh = plsc.VectorSubcoreMesh(core_axis_name="c", subcore_axis_name="s")


def _zero_rows(rows_v, nrow, ncol16):
    def body(i, carry):
        for k in range(ncol16):
            rows_v[i, pl.ds(k * 16, 16)] = jnp.zeros((16,), jnp.float32)
        return carry
    lax.fori_loop(0, nrow, body, 0)


def _spmem_zero_and_barrier(zsrc_v, shared, s, width):
    base = s * _ROWS_PER_TILE
    off = 0
    while off < _ROWS_PER_TILE:
        m = min(_CH, _ROWS_PER_TILE - off)
        pltpu.sync_copy(zsrc_v.at[pl.ds(0, m)], shared.at[pl.ds(base + off, m)])
        off += m
    plsc.subcore_barrier()


def _spmem_writeout(shared, rows_v, out_slice, s):
    base = s * _ROWS_PER_TILE
    off = 0
    while off < _ROWS_PER_TILE:
        m = min(_CH, _ROWS_PER_TILE - off)
        pltpu.sync_copy(shared.at[pl.ds(base + off, m)], rows_v.at[pl.ds(0, m)])
        pltpu.sync_copy(rows_v.at[pl.ds(0, m)], out_slice.at[pl.ds(base + off, m)])
        off += m


@functools.partial(
    pl.kernel,
    out_type=jax.ShapeDtypeStruct((_NC, _NPAD, _D), jnp.float32),
    mesh=_mesh,
    scratch_types=[
        pltpu.VMEM((_NCHUNK, _CH), jnp.int32),
        pltpu.VMEM((_NCHUNK, _CH), jnp.int32),
        pltpu.VMEM((_CH, _D), jnp.float32),
        pltpu.VMEM_SHARED((_NPAD, _D), jnp.float32),
    ],
)
def _sc_agg(h_hbm, src_hbm, dst_hbm, out_hbm, src_v, dst_v, rows, acc_sh):
    c = lax.axis_index("c")
    s = lax.axis_index("s")
    wid = c * _NS + s
    pltpu.sync_copy(src_hbm.at[wid], src_v)
    pltpu.sync_copy(dst_hbm.at[wid], dst_v)
    _zero_rows(rows, _CH, _D // 16)
    _spmem_zero_and_barrier(rows, acc_sh, s, _D)

    def body(j, carry):
        pltpu.sync_copy(h_hbm.at[src_v.at[j]], rows)
        pltpu.sync_copy(rows, acc_sh.at[dst_v.at[j]], add=True)
        return carry
    lax.fori_loop(0, _NCHUNK, body, 0)

    plsc.subcore_barrier()
    _spmem_writeout(acc_sh, rows, out_hbm.at[c], s)


@functools.partial(
    pl.kernel,
    out_type=jax.ShapeDtypeStruct((_NC, _NPAD, _D), jnp.float32),
    mesh=_mesh,
    scratch_types=[
        pltpu.VMEM((_NCHUNK, _CH), jnp.int32),
        pltpu.VMEM((_CH, _D), jnp.float32),
        pltpu.VMEM((_CH, _D), jnp.float32),
        pltpu.VMEM_SHARED((_NPAD, _D), jnp.float32),
    ],
)
def _sc_deg(dst_hbm, out_hbm, dst_v, ones_v, zero_v, acc_sh):
    c = lax.axis_index("c")
    s = lax.axis_index("s")
    wid = c * _NS + s
    pltpu.sync_copy(dst_hbm.at[wid], dst_v)

    def fill(i, carry):
        for k in range(_D // 16):
            ones_v[i, pl.ds(k * 16, 16)] = jnp.ones((16,), jnp.float32)
            zero_v[i, pl.ds(k * 16, 16)] = jnp.zeros((16,), jnp.float32)
        return carry
    lax.fori_loop(0, _CH, fill, 0)
    _spmem_zero_and_barrier(zero_v, acc_sh, s, _D)

    def body(j, carry):
        pltpu.sync_copy(ones_v, acc_sh.at[dst_v.at[j]], add=True)
        return carry
    lax.fori_loop(0, _NCHUNK, body, 0)

    plsc.subcore_barrier()
    _spmem_writeout(acc_sh, zero_v, out_hbm.at[c], s)


def _matmul(a, w):
    return lax.dot_general(
        a, w, (((1,), (0,)), ((), ())),
        precision=lax.Precision.HIGHEST,
        preferred_element_type=jnp.float32,
    )


_BM = 2000
_NBLK = _N // _BM


def _linear_body(h_ref, pp_ref, dd_ref, wl_ref, bl_ref, wr_ref,
                 t_ref, st_ref, acc_ref):
    j = pl.program_id(0)
    agg = pp_ref[0] + pp_ref[1]
    deg = dd_ref[0, :, 0:1] + dd_ref[1, :, 0:1]
    mean = agg * (1.0 / jnp.maximum(deg, 1.0))
    t = _matmul(mean, wl_ref[...]) + _matmul(h_ref[...], wr_ref[...]) \
        + bl_ref[...]
    t_ref[...] = t

    @pl.when(j == 0)
    def _():
        acc_ref[...] = jnp.zeros_like(acc_ref)

    acc_ref[0:1, :] += jnp.sum(t, axis=0, keepdims=True)
    acc_ref[1:2, :] += jnp.sum(t * t, axis=0, keepdims=True)

    @pl.when(j == _NBLK - 1)
    def _():
        mu = acc_ref[0:1, :] * (1.0 / _N)
        st_ref[0:1, :] = mu
        st_ref[1:2, :] = acc_ref[1:2, :] * (1.0 / _N) - mu * mu


def _bn_body(t_ref, st_ref, g_ref, b_ref, wc_ref, bc_ref, o_ref, *, last):
    mu = st_ref[0:1, :]
    var = st_ref[1:2, :]
    y = (t_ref[...] - mu) * lax.rsqrt(var + 1e-5) * g_ref[...] + b_ref[...]
    y = jnp.maximum(y, 0.0)
    if last:
        o_ref[...] = _matmul(y, wc_ref[...]) + bc_ref[...]
    else:
        o_ref[...] = y


def _dense_layer(h, pp, dd, Wl, bl, Wr, g, b, Wc, bc, last):
    t, st = pl.pallas_call(
        _linear_body,
        grid=(_NBLK,),
        in_specs=[
            pl.BlockSpec((_BM, _D), lambda j: (j, 0)),
            pl.BlockSpec((_NC, _BM, _D), lambda j: (0, j, 0)),
            pl.BlockSpec((_NC, _BM, 16), lambda j: (0, j, 0)),
            pl.BlockSpec((_D, _D), lambda j: (0, 0)),
            pl.BlockSpec((1, _D), lambda j: (0, 0)),
            pl.BlockSpec((_D, _D), lambda j: (0, 0)),
        ],
        out_specs=[
            pl.BlockSpec((_BM, _D), lambda j: (j, 0)),
            pl.BlockSpec((2, _D), lambda j: (0, 0)),
        ],
        out_shape=[
            jax.ShapeDtypeStruct((_N, _D), jnp.float32),
            jax.ShapeDtypeStruct((2, _D), jnp.float32),
        ],
        scratch_shapes=[pltpu.VMEM((2, _D), jnp.float32)],
    )(h, pp, dd, Wl, bl.reshape(1, -1), Wr)

    return pl.pallas_call(
        functools.partial(_bn_body, last=last),
        grid=(_NBLK,),
        in_specs=[
            pl.BlockSpec((_BM, _D), lambda j: (j, 0)),
            pl.BlockSpec((2, _D), lambda j: (0, 0)),
            pl.BlockSpec((1, _D), lambda j: (0, 0)),
            pl.BlockSpec((1, _D), lambda j: (0, 0)),
            pl.BlockSpec((_D, _D), lambda j: (0, 0)),
            pl.BlockSpec((1, _D), lambda j: (0, 0)),
        ],
        out_specs=pl.BlockSpec((_BM, _D), lambda j: (j, 0)),
        out_shape=jax.ShapeDtypeStruct((_N, _D), jnp.float32),
    )(t, st, g.reshape(1, -1), b.reshape(1, -1), Wc, bc.reshape(1, -1))


def kernel(x, edge_index, Wl0, bl0, Wr0, g0, b0, Wl1, bl1, Wr1, g1, b1,
           Wl2, bl2, Wr2, g2, b2, Wc, bc):
    src = edge_index[0]
    dst = edge_index[1]
    pad = _EPAD - _E
    src3 = jnp.concatenate([src, jnp.zeros((pad,), jnp.int32)]).reshape(
        _NW, _NCHUNK, _CH)
    dst3 = jnp.concatenate([dst, jnp.full((pad,), _N, jnp.int32)]).reshape(
        _NW, _NCHUNK, _CH)

    dd = _sc_deg(dst3)[:, :, :16]

    c = Wc.shape[1]
    Wcp = jnp.zeros((_D, 128), jnp.float32).at[:, :c].set(Wc)
    bcp = jnp.zeros((128,), jnp.float32).at[:c].set(bc)

    h = x
    params = [(Wl0, bl0, Wr0, g0, b0), (Wl1, bl1, Wr1, g1, b1),
              (Wl2, bl2, Wr2, g2, b2)]
    for i, (Wl, bl, Wr, g, b) in enumerate(params):
        pp = _sc_agg(h, src3, dst3)
        h = _dense_layer(h, pp, dd, Wl, bl, Wr, g, b, Wcp, bcp,
                         last=(i == 2))
    return h[:, :c]
